# baseline (speedup 1.0000x reference)
import functools

import jax
import jax.numpy as jnp
from jax import lax
from jax.experimental import pallas as pl
from jax.experimental.pallas import tpu as pltpu
from jax.experimental.pallas import tpu_sc as plsc

_N, _E, _D, _H, _L, _C = 10000, 320000, 128, 128, 4, 10
_NC, _NS = 2, 16
_NW = _NC * _NS
_EPW = _E // _NW
_K = 125
_NCHUNK = _EPW // _K
_HALF = _NCHUNK // 2
_HPAIR = _HALF // 2
_SZ0 = 640
_SZL = _N - 15 * _SZ0

_BR = 5000
_NBLK = _N // _BR


def _segsum_body(src_hbm, dst_hbm, h_hbm, zero_hbm, out_hbm,
                 sidx, didx, rows_a, rows_b, acc, sem_a, sem_b, zsem):
    cid = lax.axis_index("c")
    sid = lax.axis_index("s")
    wid = sid * _NC + cid
    rb = pl.multiple_of(sid * _SZ0, 8)
    ob = pl.multiple_of(cid * _N + sid * _SZ0, 8)

    @pl.when(sid < _NS - 1)
    def _():
        pltpu.async_copy(zero_hbm.at[pl.ds(rb, _SZ0)],
                         acc.at[pl.ds(rb, _SZ0)], zsem)

    @pl.when(sid == _NS - 1)
    def _():
        pltpu.async_copy(zero_hbm.at[pl.ds(rb, _SZL)],
                         acc.at[pl.ds(rb, _SZL)], zsem)

    pltpu.sync_copy(src_hbm.at[wid, 0], sidx)
    pltpu.sync_copy(dst_hbm.at[wid, 0], didx)
    pltpu.async_copy(h_hbm.at[sidx.at[0]], rows_a, sem_a)

    @pl.when(sid < _NS - 1)
    def _():
        pltpu.make_async_copy(zero_hbm.at[pl.ds(rb, _SZ0)],
                              acc.at[pl.ds(rb, _SZ0)], zsem).wait()

    @pl.when(sid == _NS - 1)
    def _():
        pltpu.make_async_copy(zero_hbm.at[pl.ds(rb, _SZL)],
                              acc.at[pl.ds(rb, _SZL)], zsem).wait()

    plsc.subcore_barrier()

    for hh in range(2):

        def pair(j, carry):
            c0 = 2 * j
            pltpu.async_copy(h_hbm.at[sidx.at[c0 + 1]], rows_b, sem_b)
            pltpu.make_async_copy(h_hbm.at[sidx.at[c0]], rows_a, sem_a).wait()
            pltpu.sync_copy(rows_a, acc.at[didx.at[c0]], add=True)

            @pl.when(j < _HPAIR - 1)
            def _():
                pltpu.async_copy(h_hbm.at[sidx.at[c0 + 2]], rows_a, sem_a)

            pltpu.make_async_copy(h_hbm.at[sidx.at[c0 + 1]], rows_b,
                                  sem_b).wait()
            pltpu.sync_copy(rows_b, acc.at[didx.at[c0 + 1]], add=True)
            return carry

        lax.fori_loop(0, _HPAIR, pair, 0)
        if hh == 0:
            pltpu.sync_copy(src_hbm.at[wid, 1], sidx)
            pltpu.sync_copy(dst_hbm.at[wid, 1], didx)
            pltpu.async_copy(h_hbm.at[sidx.at[0]], rows_a, sem_a)
    plsc.subcore_barrier()

    @pl.when(sid < _NS - 1)
    def _():
        pltpu.sync_copy(acc.at[pl.ds(rb, _SZ0)], out_hbm.at[pl.ds(ob, _SZ0)])

    @pl.when(sid == _NS - 1)
    def _():
        pltpu.sync_copy(acc.at[pl.ds(rb, _SZL)], out_hbm.at[pl.ds(ob, _SZL)])


@functools.cache
def _make_segsum():
    return pl.kernel(
        _segsum_body,
        out_type=jax.ShapeDtypeStruct((_NC * _N, _H), jnp.float32),
        mesh=plsc.VectorSubcoreMesh(core_axis_name="c", subcore_axis_name="s",
                                    num_cores=_NC, num_subcores=_NS),
        scratch_types=[
            pltpu.VMEM((_HALF, _K), jnp.int32),
            pltpu.VMEM((_HALF, _K), jnp.int32),
            pltpu.VMEM((_K, _H), jnp.float32),
            pltpu.VMEM((_K, _H), jnp.float32),
            pltpu.VMEM_SHARED((_N, _H), jnp.float32),
            pltpu.SemaphoreType.DMA,
            pltpu.SemaphoreType.DMA,
            pltpu.SemaphoreType.DMA,
        ],
    )


def _dot(a, b):
    return lax.dot_general(
        a, b, (((1,), (0,)), ((), ())),
        precision=lax.Precision.HIGHEST,
        preferred_element_type=jnp.float32)


def _emb_body(x_ref, w_ref, b_ref, out_ref, pool_ref):
    h = _dot(x_ref[...], w_ref[...]) + b_ref[...]
    out_ref[...] = h
    s = jnp.sum(h, axis=0, keepdims=True)

    @pl.when(pl.program_id(0) == 0)
    def _():
        pool_ref[...] = s

    @pl.when(pl.program_id(0) != 0)
    def _():
        pool_ref[...] += s


_emb = pl.pallas_call(
    _emb_body,
    grid=(_NBLK,),
    in_specs=[
        pl.BlockSpec((_BR, _D), lambda i: (i, 0)),
        pl.BlockSpec((_D, _H), lambda i: (0, 0)),
        pl.BlockSpec((1, _H), lambda i: (0, 0)),
    ],
    out_specs=[
        pl.BlockSpec((_BR, _H), lambda i: (i, 0)),
        pl.BlockSpec((1, _H), lambda i: (0, 0)),
    ],
    out_shape=[
        jax.ShapeDtypeStruct((_N, _H), jnp.float32),
        jax.ShapeDtypeStruct((1, _H), jnp.float32),
    ],
)


def _layer_body(h_ref, n0_ref, n1_ref, norm_ref, w1_ref, b1_ref,
                w2_ref, b2_ref, eps_ref, out_ref, pool_ref):
    h_in = h_ref[...]
    t = (1.0 + eps_ref[0, 0]) * h_in + (n0_ref[...] + n1_ref[...])
    a = jnp.maximum(
        _dot(t, w1_ref[...]) + b1_ref[...], 0.0)
    bb = _dot(a, w2_ref[...]) + b2_ref[...]
    r = jnp.maximum(bb, 0.0) * norm_ref[...]
    r = jnp.maximum(r, 0.0)
    h_out = h_in + r
    out_ref[...] = h_out
    s = jnp.sum(h_out, axis=0, keepdims=True)

    @pl.when(pl.program_id(0) == 0)
    def _():
        pool_ref[...] = s

    @pl.when(pl.program_id(0) != 0)
    def _():
        pool_ref[...] += s


_layer = pl.pallas_call(
    _layer_body,
    grid=(_NBLK,),
    in_specs=[
        pl.BlockSpec((_BR, _H), lambda i: (i, 0)),
        pl.BlockSpec((_BR, _H), lambda i: (i, 0)),
        pl.BlockSpec((_BR, _H), lambda i: (i + _NBLK, 0)),
        pl.BlockSpec((_BR, 1), lambda i: (i, 0)),
        pl.BlockSpec((_H, _H), lambda i: (0, 0)),
        pl.BlockSpec((1, _H), lambda i: (0, 0)),
        pl.BlockSpec((_H, _H), lambda i: (0, 0)),
        pl.BlockSpec((1, _H), lambda i: (0, 0)),
        pl.BlockSpec(memory_space=pltpu.SMEM),
    ],
    out_specs=[
        pl.BlockSpec((_BR, _H), lambda i: (i, 0)),
        pl.BlockSpec((1, _H), lambda i: (0, 0)),
    ],
    out_shape=[
        jax.ShapeDtypeStruct((_N, _H), jnp.float32),
        jax.ShapeDtypeStruct((1, _H), jnp.float32),
    ],
)


def _proj_body(pool_ref, wp_ref, bp_ref, out_ref):
    s = _dot(pool_ref[...], wp_ref[...])
    out_ref[...] = s + jnp.sum(bp_ref[...], axis=0, keepdims=True)


_proj = pl.pallas_call(
    _proj_body,
    out_shape=jax.ShapeDtypeStruct((1, _C), jnp.float32),
)


def kernel(edge_index, nodes_feat, edges_feat, nodes_num_norm_sqrt,
           edges_num_norm_sqrt, W_emb, b_emb, W1, b1, W2, b2, eps, Wp, bp):
    src = edge_index[0].astype(jnp.int32).reshape(_NW, 2, _HALF, _K)
    dst = edge_index[1].astype(jnp.int32).reshape(_NW, 2, _HALF, _K)
    zeros = jnp.zeros((_N, _H), jnp.float32)

    segsum = _make_segsum()
    h, pool0 = _emb(nodes_feat, W_emb, b_emb.reshape(1, _H))
    pools = [pool0]
    for i in range(_L):
        parts = segsum(src, dst, h, zeros)
        h, pool = _layer(h, parts, parts, nodes_num_norm_sqrt,
                         W1[i], b1[i].reshape(1, _H), W2[i],
                         b2[i].reshape(1, _H), eps[i].reshape(1, 1))
        pools.append(pool)

    pool_flat = jnp.concatenate(pools, axis=1)
    return _proj(pool_flat, Wp.reshape((_L + 1) * _H, _C), bp)


# speedup vs baseline: 1.1310x; 1.1310x over previous
import functools

import jax
import jax.numpy as jnp
from jax import lax
from jax.experimental import pallas as pl
from jax.experimental.pallas import tpu as pltpu
from jax.experimental.pallas import tpu_sc as plsc

_N, _E, _D, _H, _L, _C = 10000, 320000, 128, 128, 4, 10
_NC, _NS = 2, 16
_NW = _NC * _NS
_EPW = _E // _NW
_K = 125
_NCHUNK = _EPW // _K
_HALF = _NCHUNK // 2
_HPAIR = _HALF // 2
_SZ0 = 640
_SZL = _N - 15 * _SZ0

_BR = 2000
_NBLK = _N // _BR


def _segsum_body(src_hbm, dst_hbm, h_hbm, zero_hbm, out_hbm,
                 sidx, didx, rows_a, rows_b, acc, sem_a, sem_b, zsem):
    cid = lax.axis_index("c")
    sid = lax.axis_index("s")
    wid = sid * _NC + cid
    rb = pl.multiple_of(sid * _SZ0, 8)
    ob = pl.multiple_of(cid * _N + sid * _SZ0, 8)

    @pl.when(sid < _NS - 1)
    def _():
        pltpu.async_copy(zero_hbm.at[pl.ds(rb, _SZ0)],
                         acc.at[pl.ds(rb, _SZ0)], zsem)

    @pl.when(sid == _NS - 1)
    def _():
        pltpu.async_copy(zero_hbm.at[pl.ds(rb, _SZL)],
                         acc.at[pl.ds(rb, _SZL)], zsem)

    pltpu.sync_copy(src_hbm.at[wid, 0], sidx)
    pltpu.sync_copy(dst_hbm.at[wid, 0], didx)
    pltpu.async_copy(h_hbm.at[sidx.at[0]], rows_a, sem_a)

    @pl.when(sid < _NS - 1)
    def _():
        pltpu.make_async_copy(zero_hbm.at[pl.ds(rb, _SZ0)],
                              acc.at[pl.ds(rb, _SZ0)], zsem).wait()

    @pl.when(sid == _NS - 1)
    def _():
        pltpu.make_async_copy(zero_hbm.at[pl.ds(rb, _SZL)],
                              acc.at[pl.ds(rb, _SZL)], zsem).wait()

    plsc.subcore_barrier()

    for hh in range(2):

        def pair(j, carry):
            c0 = 2 * j
            pltpu.async_copy(h_hbm.at[sidx.at[c0 + 1]], rows_b, sem_b)
            pltpu.make_async_copy(h_hbm.at[sidx.at[c0]], rows_a, sem_a).wait()
            pltpu.sync_copy(rows_a, acc.at[didx.at[c0]], add=True)

            @pl.when(j < _HPAIR - 1)
            def _():
                pltpu.async_copy(h_hbm.at[sidx.at[c0 + 2]], rows_a, sem_a)

            pltpu.make_async_copy(h_hbm.at[sidx.at[c0 + 1]], rows_b,
                                  sem_b).wait()
            pltpu.sync_copy(rows_b, acc.at[didx.at[c0 + 1]], add=True)
            return carry

        lax.fori_loop(0, _HPAIR, pair, 0)
        if hh == 0:
            pltpu.sync_copy(src_hbm.at[wid, 1], sidx)
            pltpu.sync_copy(dst_hbm.at[wid, 1], didx)
            pltpu.async_copy(h_hbm.at[sidx.at[0]], rows_a, sem_a)
    plsc.subcore_barrier()

    @pl.when(sid < _NS - 1)
    def _():
        pltpu.sync_copy(acc.at[pl.ds(rb, _SZ0)], out_hbm.at[pl.ds(ob, _SZ0)])

    @pl.when(sid == _NS - 1)
    def _():
        pltpu.sync_copy(acc.at[pl.ds(rb, _SZL)], out_hbm.at[pl.ds(ob, _SZL)])


@functools.cache
def _make_segsum():
    return pl.kernel(
        _segsum_body,
        out_type=jax.ShapeDtypeStruct((_NC * _N, _H), jnp.float32),
        mesh=plsc.VectorSubcoreMesh(core_axis_name="c", subcore_axis_name="s",
                                    num_cores=_NC, num_subcores=_NS),
        scratch_types=[
            pltpu.VMEM((_HALF, _K), jnp.int32),
            pltpu.VMEM((_HALF, _K), jnp.int32),
            pltpu.VMEM((_K, _H), jnp.float32),
            pltpu.VMEM((_K, _H), jnp.float32),
            pltpu.VMEM_SHARED((_N, _H), jnp.float32),
            pltpu.SemaphoreType.DMA,
            pltpu.SemaphoreType.DMA,
            pltpu.SemaphoreType.DMA,
        ],
    )


def _dot(a, b):
    return lax.dot_general(
        a, b, (((1,), (0,)), ((), ())),
        precision=lax.Precision.HIGHEST,
        preferred_element_type=jnp.float32)


def _emb_body(x_ref, w_ref, b_ref, out_ref, pool_ref):
    h = _dot(x_ref[...], w_ref[...]) + b_ref[...]
    out_ref[...] = h
    s = jnp.sum(h, axis=0, keepdims=True)

    @pl.when(pl.program_id(0) == 0)
    def _():
        pool_ref[...] = s

    @pl.when(pl.program_id(0) != 0)
    def _():
        pool_ref[...] += s


_emb = pl.pallas_call(
    _emb_body,
    grid=(_NBLK,),
    in_specs=[
        pl.BlockSpec((_BR, _D), lambda i: (i, 0)),
        pl.BlockSpec((_D, _H), lambda i: (0, 0)),
        pl.BlockSpec((1, _H), lambda i: (0, 0)),
    ],
    out_specs=[
        pl.BlockSpec((_BR, _H), lambda i: (i, 0)),
        pl.BlockSpec((1, _H), lambda i: (0, 0)),
    ],
    out_shape=[
        jax.ShapeDtypeStruct((_N, _H), jnp.float32),
        jax.ShapeDtypeStruct((1, _H), jnp.float32),
    ],
)


def _layer_body(h_ref, n0_ref, n1_ref, norm_ref, w1_ref, b1_ref,
                w2_ref, b2_ref, eps_ref, out_ref, pool_ref):
    h_in = h_ref[...]
    t = (1.0 + eps_ref[0, 0]) * h_in + (n0_ref[...] + n1_ref[...])
    a = jnp.maximum(
        _dot(t, w1_ref[...]) + b1_ref[...], 0.0)
    bb = _dot(a, w2_ref[...]) + b2_ref[...]
    r = jnp.maximum(bb, 0.0) * norm_ref[...]
    r = jnp.maximum(r, 0.0)
    h_out = h_in + r
    out_ref[...] = h_out
    s = jnp.sum(h_out, axis=0, keepdims=True)

    @pl.when(pl.program_id(0) == 0)
    def _():
        pool_ref[...] = s

    @pl.when(pl.program_id(0) != 0)
    def _():
        pool_ref[...] += s


_layer = pl.pallas_call(
    _layer_body,
    grid=(_NBLK,),
    in_specs=[
        pl.BlockSpec((_BR, _H), lambda i: (i, 0)),
        pl.BlockSpec((_BR, _H), lambda i: (i, 0)),
        pl.BlockSpec((_BR, _H), lambda i: (i + _NBLK, 0)),
        pl.BlockSpec((_BR, 1), lambda i: (i, 0)),
        pl.BlockSpec((_H, _H), lambda i: (0, 0)),
        pl.BlockSpec((1, _H), lambda i: (0, 0)),
        pl.BlockSpec((_H, _H), lambda i: (0, 0)),
        pl.BlockSpec((1, _H), lambda i: (0, 0)),
        pl.BlockSpec(memory_space=pltpu.SMEM),
    ],
    out_specs=[
        pl.BlockSpec((_BR, _H), lambda i: (i, 0)),
        pl.BlockSpec((1, _H), lambda i: (0, 0)),
    ],
    out_shape=[
        jax.ShapeDtypeStruct((_N, _H), jnp.float32),
        jax.ShapeDtypeStruct((1, _H), jnp.float32),
    ],
)


def _proj_body(pool_ref, wp_ref, bp_ref, out_ref):
    s = _dot(pool_ref[...], wp_ref[...])
    out_ref[...] = s + jnp.sum(bp_ref[...], axis=0, keepdims=True)


_proj = pl.pallas_call(
    _proj_body,
    out_shape=jax.ShapeDtypeStruct((1, _C), jnp.float32),
)


def kernel(edge_index, nodes_feat, edges_feat, nodes_num_norm_sqrt,
           edges_num_norm_sqrt, W_emb, b_emb, W1, b1, W2, b2, eps, Wp, bp):
    src = edge_index[0].astype(jnp.int32).reshape(_NW, 2, _HALF, _K)
    dst = edge_index[1].astype(jnp.int32).reshape(_NW, 2, _HALF, _K)
    zeros = jnp.zeros((_N, _H), jnp.float32)

    segsum = _make_segsum()
    h, pool0 = _emb(nodes_feat, W_emb, b_emb.reshape(1, _H))
    pools = [pool0]
    for i in range(_L):
        parts = segsum(src, dst, h, zeros)
        h, pool = _layer(h, parts, parts, nodes_num_norm_sqrt,
                         W1[i], b1[i].reshape(1, _H), W2[i],
                         b2[i].reshape(1, _H), eps[i].reshape(1, 1))
        pools.append(pool)

    pool_flat = jnp.concatenate(pools, axis=1)
    return _proj(pool_flat, Wp.reshape((_L + 1) * _H, _C), bp)


# speedup vs baseline: 1.1718x; 1.0361x over previous
import functools

import jax
import jax.numpy as jnp
from jax import lax
from jax.experimental import pallas as pl
from jax.experimental.pallas import tpu as pltpu
from jax.experimental.pallas import tpu_sc as plsc

_N, _E, _D, _H, _L, _C = 10000, 320000, 128, 128, 4, 10
_NC, _NS = 2, 16
_NW = _NC * _NS
_EPW = _E // _NW
_K = 125
_NCHUNK = _EPW // _K
_HALF = _NCHUNK // 2
_HPAIR = _HALF // 2
_SZ0 = 640
_SZL = _N - 15 * _SZ0

_BR = 2000
_NBLK = _N // _BR


def _segsum_body(src_hbm, dst_hbm, h_hbm, zero_hbm, out_hbm,
                 sidx, didx, rows_a, rows_b, acc, sem_a, sem_b, zsem):
    cid = lax.axis_index("c")
    sid = lax.axis_index("s")
    wid = sid * _NC + cid
    rb = pl.multiple_of(sid * _SZ0, 8)
    ob = pl.multiple_of(cid * _N + sid * _SZ0, 8)

    @pl.when(sid < _NS - 1)
    def _():
        pltpu.async_copy(zero_hbm.at[pl.ds(rb, _SZ0)],
                         acc.at[pl.ds(rb, _SZ0)], zsem)

    @pl.when(sid == _NS - 1)
    def _():
        pltpu.async_copy(zero_hbm.at[pl.ds(rb, _SZL)],
                         acc.at[pl.ds(rb, _SZL)], zsem)

    pltpu.sync_copy(src_hbm.at[wid, 0], sidx)
    pltpu.sync_copy(dst_hbm.at[wid, 0], didx)
    pltpu.async_copy(h_hbm.at[sidx.at[0]], rows_a, sem_a)

    @pl.when(sid < _NS - 1)
    def _():
        pltpu.make_async_copy(zero_hbm.at[pl.ds(rb, _SZ0)],
                              acc.at[pl.ds(rb, _SZ0)], zsem).wait()

    @pl.when(sid == _NS - 1)
    def _():
        pltpu.make_async_copy(zero_hbm.at[pl.ds(rb, _SZL)],
                              acc.at[pl.ds(rb, _SZL)], zsem).wait()

    plsc.subcore_barrier()

    for hh in range(2):

        def pair(j, carry):
            c0 = 2 * j
            pltpu.async_copy(h_hbm.at[sidx.at[c0 + 1]], rows_b, sem_b)
            pltpu.make_async_copy(h_hbm.at[sidx.at[c0]], rows_a, sem_a).wait()
            pltpu.sync_copy(rows_a, acc.at[didx.at[c0]], add=True)

            @pl.when(j < _HPAIR - 1)
            def _():
                pltpu.async_copy(h_hbm.at[sidx.at[c0 + 2]], rows_a, sem_a)

            pltpu.make_async_copy(h_hbm.at[sidx.at[c0 + 1]], rows_b,
                                  sem_b).wait()
            pltpu.sync_copy(rows_b, acc.at[didx.at[c0 + 1]], add=True)
            return carry

        lax.fori_loop(0, _HPAIR, pair, 0)
        if hh == 0:
            pltpu.sync_copy(src_hbm.at[wid, 1], sidx)
            pltpu.sync_copy(dst_hbm.at[wid, 1], didx)
            pltpu.async_copy(h_hbm.at[sidx.at[0]], rows_a, sem_a)
    plsc.subcore_barrier()

    @pl.when(sid < _NS - 1)
    def _():
        pltpu.sync_copy(acc.at[pl.ds(rb, _SZ0)], out_hbm.at[pl.ds(ob, _SZ0)])

    @pl.when(sid == _NS - 1)
    def _():
        pltpu.sync_copy(acc.at[pl.ds(rb, _SZL)], out_hbm.at[pl.ds(ob, _SZL)])


@functools.cache
def _make_segsum():
    return pl.kernel(
        _segsum_body,
        out_type=jax.ShapeDtypeStruct((_NC * _N, _H), jnp.float32),
        mesh=plsc.VectorSubcoreMesh(core_axis_name="c", subcore_axis_name="s",
                                    num_cores=_NC, num_subcores=_NS),
        scratch_types=[
            pltpu.VMEM((_HALF, _K), jnp.int32),
            pltpu.VMEM((_HALF, _K), jnp.int32),
            pltpu.VMEM((_K, _H), jnp.float32),
            pltpu.VMEM((_K, _H), jnp.float32),
            pltpu.VMEM_SHARED((_N, _H), jnp.float32),
            pltpu.SemaphoreType.DMA,
            pltpu.SemaphoreType.DMA,
            pltpu.SemaphoreType.DMA,
        ],
    )


def _dot(a, b):
    a_hi = a.astype(jnp.bfloat16)
    a_lo = (a - a_hi.astype(jnp.float32)).astype(jnp.bfloat16)
    b_hi = b.astype(jnp.bfloat16)
    b_lo = (b - b_hi.astype(jnp.float32)).astype(jnp.bfloat16)

    def dg(x, y):
        return lax.dot_general(x, y, (((1,), (0,)), ((), ())),
                               preferred_element_type=jnp.float32)

    return dg(a_hi, b_hi) + (dg(a_hi, b_lo) + dg(a_lo, b_hi))


def _emb_body(x_ref, w_ref, b_ref, out_ref, pool_ref):
    h = _dot(x_ref[...], w_ref[...]) + b_ref[...]
    out_ref[...] = h
    s = jnp.sum(h, axis=0, keepdims=True)

    @pl.when(pl.program_id(0) == 0)
    def _():
        pool_ref[...] = s

    @pl.when(pl.program_id(0) != 0)
    def _():
        pool_ref[...] += s


_emb = pl.pallas_call(
    _emb_body,
    grid=(_NBLK,),
    in_specs=[
        pl.BlockSpec((_BR, _D), lambda i: (i, 0)),
        pl.BlockSpec((_D, _H), lambda i: (0, 0)),
        pl.BlockSpec((1, _H), lambda i: (0, 0)),
    ],
    out_specs=[
        pl.BlockSpec((_BR, _H), lambda i: (i, 0)),
        pl.BlockSpec((1, _H), lambda i: (0, 0)),
    ],
    out_shape=[
        jax.ShapeDtypeStruct((_N, _H), jnp.float32),
        jax.ShapeDtypeStruct((1, _H), jnp.float32),
    ],
)


def _layer_body(h_ref, n0_ref, n1_ref, norm_ref, w1_ref, b1_ref,
                w2_ref, b2_ref, eps_ref, out_ref, pool_ref):
    h_in = h_ref[...]
    t = (1.0 + eps_ref[0, 0]) * h_in + (n0_ref[...] + n1_ref[...])
    a = jnp.maximum(
        _dot(t, w1_ref[...]) + b1_ref[...], 0.0)
    bb = _dot(a, w2_ref[...]) + b2_ref[...]
    r = jnp.maximum(bb, 0.0) * norm_ref[...]
    r = jnp.maximum(r, 0.0)
    h_out = h_in + r
    out_ref[...] = h_out
    s = jnp.sum(h_out, axis=0, keepdims=True)

    @pl.when(pl.program_id(0) == 0)
    def _():
        pool_ref[...] = s

    @pl.when(pl.program_id(0) != 0)
    def _():
        pool_ref[...] += s


_layer = pl.pallas_call(
    _layer_body,
    grid=(_NBLK,),
    in_specs=[
        pl.BlockSpec((_BR, _H), lambda i: (i, 0)),
        pl.BlockSpec((_BR, _H), lambda i: (i, 0)),
        pl.BlockSpec((_BR, _H), lambda i: (i + _NBLK, 0)),
        pl.BlockSpec((_BR, 1), lambda i: (i, 0)),
        pl.BlockSpec((_H, _H), lambda i: (0, 0)),
        pl.BlockSpec((1, _H), lambda i: (0, 0)),
        pl.BlockSpec((_H, _H), lambda i: (0, 0)),
        pl.BlockSpec((1, _H), lambda i: (0, 0)),
        pl.BlockSpec(memory_space=pltpu.SMEM),
    ],
    out_specs=[
        pl.BlockSpec((_BR, _H), lambda i: (i, 0)),
        pl.BlockSpec((1, _H), lambda i: (0, 0)),
    ],
    out_shape=[
        jax.ShapeDtypeStruct((_N, _H), jnp.float32),
        jax.ShapeDtypeStruct((1, _H), jnp.float32),
    ],
)


def _proj_body(pool_ref, wp_ref, bp_ref, out_ref):
    s = _dot(pool_ref[...], wp_ref[...])
    out_ref[...] = s + jnp.sum(bp_ref[...], axis=0, keepdims=True)


_proj = pl.pallas_call(
    _proj_body,
    out_shape=jax.ShapeDtypeStruct((1, _C), jnp.float32),
)


def kernel(edge_index, nodes_feat, edges_feat, nodes_num_norm_sqrt,
           edges_num_norm_sqrt, W_emb, b_emb, W1, b1, W2, b2, eps, Wp, bp):
    src = edge_index[0].astype(jnp.int32).reshape(_NW, 2, _HALF, _K)
    dst = edge_index[1].astype(jnp.int32).reshape(_NW, 2, _HALF, _K)
    zeros = jnp.zeros((_N, _H), jnp.float32)

    segsum = _make_segsum()
    h, pool0 = _emb(nodes_feat, W_emb, b_emb.reshape(1, _H))
    pools = [pool0]
    for i in range(_L):
        parts = segsum(src, dst, h, zeros)
        h, pool = _layer(h, parts, parts, nodes_num_norm_sqrt,
                         W1[i], b1[i].reshape(1, _H), W2[i],
                         b2[i].reshape(1, _H), eps[i].reshape(1, 1))
        pools.append(pool)

    pool_flat = jnp.concatenate(pools, axis=1)
    return _proj(pool_flat, Wp.reshape((_L + 1) * _H, _C), bp)
